# Initial kernel scaffold; baseline (speedup 1.0000x reference)
import jax
import jax.numpy as jnp
from jax.experimental import pallas as pl


def kernel(img):
    raise NotImplementedError("write your pallas kernel here")


# speedup vs baseline: 2.6033x; 2.6033x over previous
import functools

import jax
import jax.numpy as jnp
import numpy as np
from jax.experimental import pallas as pl

MIN_DISABLED = 2
MAX_DISABLED = 8
NEIGHBOR_PROB = 0.5


def _disabled_tofs(tof_count):
    rng = np.random.default_rng(0)
    disabled_count = int(rng.integers(MIN_DISABLED, MAX_DISABLED + 1))
    initial = int(rng.integers(0, tof_count))
    disabled = [initial]
    tof_list = [int(t) for t in rng.permutation(tof_count) if int(t) != initial]
    for _ in range(disabled_count - 1):
        rv = float(rng.random())
        perm = rng.permutation(len(disabled))
        permuted = [disabled[int(j)] for j in perm]
        if rv < NEIGHBOR_PROB:
            if rv < NEIGHBOR_PROB / 2:
                for cur in permuted:
                    new_neighbor = (cur + 1) % tof_count
                    if new_neighbor not in disabled:
                        disabled.append(new_neighbor)
                        tof_list = [t for t in tof_list if t != new_neighbor]
                        break
            else:
                opposite_found = False
                for cur in permuted:
                    new_opposite = (cur + tof_count // 2) % tof_count
                    if new_opposite not in disabled:
                        disabled.append(new_opposite)
                        tof_list = [t for t in tof_list if t != new_opposite]
                        opposite_found = True
                        break
                if not opposite_found:
                    new_element = tof_list[0]
                    tof_list = [t for t in tof_list if t != new_element]
                    disabled.append(new_element)
        else:
            new_element = tof_list[0]
            tof_list = [t for t in tof_list if t != new_element]
            disabled.append(new_element)
    return tuple(sorted(int(t) for t in disabled))


def _mask_body(mask_ref, x_ref, o_ref):
    o_ref[...] = x_ref[...] * mask_ref[...]


@functools.partial(jax.jit, static_argnames=("block_rows",))
def kernel(img, *, block_rows=2048):
    rows, cols = img.shape
    disabled = _disabled_tofs(cols)
    mask = np.ones((1, cols), dtype=img.dtype)
    mask[0, list(disabled)] = 0.0
    grid = (rows // block_rows,)
    return pl.pallas_call(
        _mask_body,
        grid=grid,
        in_specs=[
            pl.BlockSpec((1, cols), lambda i: (0, 0)),
            pl.BlockSpec((block_rows, cols), lambda i: (i, 0)),
        ],
        out_specs=pl.BlockSpec((block_rows, cols), lambda i: (i, 0)),
        out_shape=jax.ShapeDtypeStruct((rows, cols), img.dtype),
    )(jnp.asarray(mask), img)


# speedup vs baseline: 2.6609x; 1.0221x over previous
import functools

import jax
import jax.numpy as jnp
import numpy as np
from jax.experimental import pallas as pl

MIN_DISABLED = 2
MAX_DISABLED = 8
NEIGHBOR_PROB = 0.5


def _disabled_tofs(tof_count):
    rng = np.random.default_rng(0)
    disabled_count = int(rng.integers(MIN_DISABLED, MAX_DISABLED + 1))
    initial = int(rng.integers(0, tof_count))
    disabled = [initial]
    tof_list = [int(t) for t in rng.permutation(tof_count) if int(t) != initial]
    for _ in range(disabled_count - 1):
        rv = float(rng.random())
        perm = rng.permutation(len(disabled))
        permuted = [disabled[int(j)] for j in perm]
        if rv < NEIGHBOR_PROB:
            if rv < NEIGHBOR_PROB / 2:
                for cur in permuted:
                    new_neighbor = (cur + 1) % tof_count
                    if new_neighbor not in disabled:
                        disabled.append(new_neighbor)
                        tof_list = [t for t in tof_list if t != new_neighbor]
                        break
            else:
                opposite_found = False
                for cur in permuted:
                    new_opposite = (cur + tof_count // 2) % tof_count
                    if new_opposite not in disabled:
                        disabled.append(new_opposite)
                        tof_list = [t for t in tof_list if t != new_opposite]
                        opposite_found = True
                        break
                if not opposite_found:
                    new_element = tof_list[0]
                    tof_list = [t for t in tof_list if t != new_element]
                    disabled.append(new_element)
        else:
            new_element = tof_list[0]
            tof_list = [t for t in tof_list if t != new_element]
            disabled.append(new_element)
    return tuple(sorted(int(t) for t in disabled))


def _mask_body(mask_ref, x_ref, o_ref):
    o_ref[...] = x_ref[...] * mask_ref[...]


@functools.partial(jax.jit, static_argnames=("block_rows",))
def kernel(img, *, block_rows=4096):
    rows, cols = img.shape
    disabled = _disabled_tofs(cols)
    mask = np.ones((1, cols), dtype=img.dtype)
    mask[0, list(disabled)] = 0.0
    grid = (rows // block_rows,)
    return pl.pallas_call(
        _mask_body,
        grid=grid,
        in_specs=[
            pl.BlockSpec((1, cols), lambda i: (0, 0)),
            pl.BlockSpec((block_rows, cols), lambda i: (i, 0)),
        ],
        out_specs=pl.BlockSpec((block_rows, cols), lambda i: (i, 0)),
        out_shape=jax.ShapeDtypeStruct((rows, cols), img.dtype),
    )(jnp.asarray(mask), img)
